# Initial kernel scaffold; baseline (speedup 1.0000x reference)
import jax
import jax.numpy as jnp
from jax.experimental import pallas as pl


def kernel(x, up_index, down_index):
    raise NotImplementedError("write your pallas kernel here")


# speedup vs baseline: 2.6953x; 2.6953x over previous
import functools

import jax
import jax.numpy as jnp
from jax import lax
from jax.experimental import pallas as pl
from jax.experimental.pallas import tpu as pltpu
from jax.experimental.pallas import tpu_sc as plsc

N_NODES = 10000
D_FEAT = 256
DH = 128
E_RAW = 320000
BLK = 128
N_TILES = 16
BLOCKS_PER_TILE = 160
E_PAD = N_TILES * BLOCKS_PER_TILE * BLK
ACC_ROWS = 10240
ZERO_BLKS = ACC_ROWS // (N_TILES * BLK)
ROWS_OUT = 624
ROWS_OUT_LAST = 640
CHUNK = 16
N_CHUNKS = BLOCKS_PER_TILE // CHUNK

_mesh = plsc.VectorSubcoreMesh(core_axis_name="c", subcore_axis_name="s")


@functools.partial(
    pl.kernel,
    mesh=_mesh,
    out_type=(
        jax.ShapeDtypeStruct((N_NODES, DH), jnp.float32),
        jax.ShapeDtypeStruct((N_NODES, DH), jnp.float32),
    ),
    scratch_types=[
        pltpu.VMEM((CHUNK, BLK), jnp.int32),
        pltpu.VMEM((CHUNK, BLK), jnp.int32),
        pltpu.VMEM((BLK, DH), jnp.float32),
        pltpu.VMEM_SHARED((ACC_ROWS, DH), jnp.float32),
        pltpu.SemaphoreType.DMA,
    ],
)
def _mp_kernel(x_lo, x_hi, src_hbm, dst_hbm, out_lo, out_hi,
               src_v, dst_v, rows_v, acc, sem):
    c = lax.axis_index("c")
    s = lax.axis_index("s")

    zeros16 = jnp.zeros((16,), jnp.float32)

    def zrow(i, carry):
        r = i // (DH // 16)
        k = i % (DH // 16)
        rows_v[r, pl.ds(k * 16, 16)] = zeros16
        return carry

    lax.fori_loop(0, BLK * (DH // 16), zrow, 0)
    for z in range(ZERO_BLKS):
        pltpu.sync_copy(rows_v, acc.at[pl.ds((s * ZERO_BLKS + z) * BLK, BLK)])

    base = s * BLOCKS_PER_TILE

    plsc.subcore_barrier()

    def edge_loop(x_ref):
        def chunk_body(ch, carry):
            pltpu.sync_copy(src_hbm.at[pl.ds(base + ch * CHUNK, CHUNK)], src_v)
            pltpu.sync_copy(dst_hbm.at[pl.ds(base + ch * CHUNK, CHUNK)], dst_v)

            def body(b, inner):
                pltpu.async_copy(x_ref.at[src_v.at[b]], rows_v, sem).wait()
                pltpu.sync_copy(rows_v, acc.at[dst_v.at[b]], add=True)
                return inner
            lax.fori_loop(0, CHUNK, body, 0)
            return carry
        lax.fori_loop(0, N_CHUNKS, chunk_body, 0)

    @pl.when(c == 0)
    def _():
        edge_loop(x_lo)

    @pl.when(c == 1)
    def _():
        edge_loop(x_hi)

    plsc.subcore_barrier()

    rbase = s * ROWS_OUT

    def copy_out(out_ref):
        @pl.when(s < N_TILES - 1)
        def _():
            pltpu.sync_copy(acc.at[pl.ds(rbase, ROWS_OUT)],
                            out_ref.at[pl.ds(rbase, ROWS_OUT)])

        @pl.when(s == N_TILES - 1)
        def _():
            pltpu.sync_copy(acc.at[pl.ds(rbase, ROWS_OUT_LAST)],
                            out_ref.at[pl.ds(rbase, ROWS_OUT_LAST)])

    @pl.when(c == 0)
    def _():
        copy_out(out_lo)

    @pl.when(c == 1)
    def _():
        copy_out(out_hi)


def kernel(x, up_index, down_index):
    src = jnp.concatenate([up_index[0], down_index[0]]).astype(jnp.int32)
    dst = jnp.concatenate([up_index[1], down_index[1]]).astype(jnp.int32)
    pad = E_PAD - E_RAW
    src = jnp.concatenate([src, jnp.zeros((pad,), jnp.int32)])
    dst = jnp.concatenate([dst, jnp.full((pad,), N_NODES, jnp.int32)])
    src2d = src.reshape(N_TILES * BLOCKS_PER_TILE, BLK)
    dst2d = dst.reshape(N_TILES * BLOCKS_PER_TILE, BLK)
    x_lo = x[:, :DH]
    x_hi = x[:, DH:]
    out_lo, out_hi = _mp_kernel(x_lo, x_hi, src2d, dst2d)
    return jnp.concatenate([out_lo, out_hi], axis=1)
